# baseline (speedup 1.0000x reference)
import functools

import jax
import jax.numpy as jnp
from jax import lax
from jax.experimental import pallas as pl
from jax.experimental.pallas import tpu as pltpu
from jax.experimental.pallas import tpu_sc as plsc

N = 65536
H = 64
D = 64
K = 1024
EPS = 1e-5

BLK = 512
NUM_BLKS = N // BLK

SC_NC = 2
SC_NS = 16
SC_NW = SC_NC * SC_NS
SC_BPW = N // SC_NW
SC_CH = 256


def _fold_sum(x):
    w = x.shape[-1]
    while w > 1:
        w //= 2
        x = x[:, :w] + x[:, w:2 * w]
    return x


def _group_sum(x):
    acc = x[:, 0:8]
    for v in range(1, 8):
        acc = acc + x[:, 8 * v:8 * v + 8]
    acc = acc[:, 0:4] + acc[:, 4:8]
    acc = acc[:, 0:2] + acc[:, 2:4]
    return acc[:, 0:1] + acc[:, 1:2]


def _tc_body(nxy_ref, w1_ref, b1_ref, g_ref, be_ref, w2_ref, b2_ref,
             cbt_ref, cb_ref, idx_ref, loss_ref, csq_ref):
    i = pl.program_id(0)

    @pl.when(i == 0)
    def _():
        csq_ref[...] = _group_sum(cb_ref[...] ** 2).reshape(1, K)

    nxy = nxy_ref[...]
    w1 = w1_ref[...]
    nb = nxy.astype(jnp.bfloat16).astype(jnp.float32)
    wb = w1.astype(jnp.bfloat16).astype(jnp.float32)
    h = (nb[:, 0:1] * wb[0:1, :] + nb[:, 1:2] * wb[1:2, :]) + b1_ref[...]
    mu = _fold_sum(h) / float(H)
    var = _group_sum((h - mu) ** 2) / float(H)
    hn = (h - mu) / jnp.sqrt(var + EPS) * g_ref[...] + be_ref[...]
    hr = jnp.maximum(hn, 0.0)
    z = jnp.dot(hr.astype(jnp.bfloat16), w2_ref[...],
                preferred_element_type=jnp.float32) + b2_ref[...]
    zsq = _group_sum(z ** 2)
    dots = jnp.dot(z.astype(jnp.bfloat16), cbt_ref[...],
                   preferred_element_type=jnp.float32)
    dist = zsq - 2.0 * dots + csq_ref[...]
    minv = jnp.min(dist, axis=1)
    iota = jax.lax.broadcasted_iota(jnp.int32, dist.shape, 1)
    idx = jnp.min(jnp.where(dist == minv[:, None], iota, K), axis=1)
    idx_ref[...] = idx[:, None]
    s = jnp.sum(minv).reshape(1, 1)

    @pl.when(i == 0)
    def _():
        loss_ref[...] = jnp.zeros_like(loss_ref)

    loss_ref[...] += s


_tc_call = pl.pallas_call(
    _tc_body,
    grid=(NUM_BLKS,),
    in_specs=[
        pl.BlockSpec((BLK, 2), lambda i: (i, 0)),
        pl.BlockSpec((2, H), lambda i: (0, 0)),
        pl.BlockSpec((1, H), lambda i: (0, 0)),
        pl.BlockSpec((1, H), lambda i: (0, 0)),
        pl.BlockSpec((1, H), lambda i: (0, 0)),
        pl.BlockSpec((H, D), lambda i: (0, 0)),
        pl.BlockSpec((1, D), lambda i: (0, 0)),
        pl.BlockSpec((D, K), lambda i: (0, 0)),
        pl.BlockSpec((K, D), lambda i: (0, 0)),
    ],
    out_specs=[
        pl.BlockSpec((BLK, 1), lambda i: (i, 0)),
        pl.BlockSpec((1, 1), lambda i: (0, 0)),
    ],
    out_shape=[
        jax.ShapeDtypeStruct((N, 1), jnp.int32),
        jax.ShapeDtypeStruct((1, 1), jnp.float32),
    ],
    scratch_shapes=[pltpu.VMEM((1, K), jnp.float32)],
)


SC_NCH = SC_BPW // SC_CH
SC_NBUF = 3


def _sc_gather_body(cb_hbm, idx_hbm, out_hbm, idx_v, b0, b1, b2,
                    g0, g1, g2, o0, o1, o2):
    wid = lax.axis_index("s") * SC_NC + lax.axis_index("c")
    base = wid * SC_BPW
    bufs = (b0, b1, b2)
    gsem = (g0, g1, g2)
    osem = (o0, o1, o2)

    pltpu.sync_copy(idx_hbm.at[pl.ds(base, SC_BPW)], idx_v)

    def gather(g):
        return pltpu.async_copy(
            cb_hbm.at[idx_v.at[pl.ds(g * SC_CH, SC_CH)]],
            bufs[g % SC_NBUF], gsem[g % SC_NBUF])

    hg = {}
    ho = {}
    for g in range(SC_NBUF):
        hg[g] = gather(g)
    for g in range(SC_NCH):
        hg[g].wait()
        ho[g] = pltpu.async_copy(
            bufs[g % SC_NBUF],
            out_hbm.at[pl.ds(base + g * SC_CH, SC_CH)],
            osem[g % SC_NBUF])
        ng = g + SC_NBUF
        if ng < SC_NCH:
            ho[g].wait()
            hg[ng] = gather(ng)
    for g in range(SC_NCH - SC_NBUF, SC_NCH):
        if g >= 0 and g in ho and (g + SC_NBUF) >= SC_NCH:
            ho[g].wait()


@functools.lru_cache(maxsize=1)
def _make_sc_gather():
    return functools.partial(
        pl.kernel,
        mesh=plsc.VectorSubcoreMesh(core_axis_name="c", subcore_axis_name="s"),
        out_type=jax.ShapeDtypeStruct((N, 128), jnp.float32),
        scratch_types=[
            pltpu.VMEM((SC_BPW,), jnp.int32),
            pltpu.VMEM((SC_CH, 128), jnp.float32),
            pltpu.VMEM((SC_CH, 128), jnp.float32),
            pltpu.VMEM((SC_CH, 128), jnp.float32),
            pltpu.SemaphoreType.DMA,
            pltpu.SemaphoreType.DMA,
            pltpu.SemaphoreType.DMA,
            pltpu.SemaphoreType.DMA,
            pltpu.SemaphoreType.DMA,
            pltpu.SemaphoreType.DMA,
        ],
    )(_sc_gather_body)


def kernel(xy, W1, b1, gamma, beta, W2, b2, codebook):
    f = xy.astype(jnp.float32)
    nx = f[:, 0] / 511.0 * 2.0 - 1.0
    ny = f[:, 1] / 511.0 * 2.0 - 1.0
    norm_xy = jnp.stack([nx, ny], axis=1)

    idx2d, loss_sum = _tc_call(
        norm_xy, W1, b1.reshape(1, H), gamma.reshape(1, H), beta.reshape(1, H),
        W2.astype(jnp.bfloat16), b2.reshape(1, D),
        codebook.T.astype(jnp.bfloat16), codebook)

    cb_pad = jnp.concatenate(
        [codebook, jnp.zeros((K, 128 - D), jnp.float32)], axis=1)
    q = _make_sc_gather()(cb_pad, idx2d.reshape(N))[:, :D]

    l = loss_sum[0, 0] / (N * D)
    loss = l + 0.25 * l
    return (q, idx2d, loss)


# speedup vs baseline: 2.0460x; 2.0460x over previous
import functools

import jax
import jax.numpy as jnp
from jax import lax
from jax.experimental import pallas as pl
from jax.experimental.pallas import tpu as pltpu
from jax.experimental.pallas import tpu_sc as plsc

N = 65536
H = 64
D = 64
K = 1024
EPS = 1e-5

BLK = 512
NUM_BLKS = N // BLK

SC_NC = 2
SC_NS = 16
SC_NW = SC_NC * SC_NS
SC_BPW = N // SC_NW
SC_CH = 256


def _fold_sum(x):
    w = x.shape[-1]
    while w > 1:
        w //= 2
        x = x[:, :w] + x[:, w:2 * w]
    return x


def _group_sum(x):
    acc = x[:, 0:8]
    for v in range(1, 8):
        acc = acc + x[:, 8 * v:8 * v + 8]
    acc = acc[:, 0:4] + acc[:, 4:8]
    acc = acc[:, 0:2] + acc[:, 2:4]
    return acc[:, 0:1] + acc[:, 1:2]


def _tc_body(nxy_ref, w1_ref, b1_ref, g_ref, be_ref, w2_ref, b2_ref,
             cbt_ref, cb_ref, idx_ref, loss_ref, csq_ref):
    i = pl.program_id(0)

    @pl.when(i == 0)
    def _():
        csq_ref[...] = _group_sum(cb_ref[...] ** 2).reshape(1, K)

    nxy = nxy_ref[...]
    w1 = w1_ref[...]
    nb = nxy.astype(jnp.bfloat16).astype(jnp.float32)
    wb = w1.astype(jnp.bfloat16).astype(jnp.float32)
    h = (nb[:, 0:1] * wb[0:1, :] + nb[:, 1:2] * wb[1:2, :]) + b1_ref[...]
    mu = _fold_sum(h) / float(H)
    var = _group_sum((h - mu) ** 2) / float(H)
    hn = (h - mu) / jnp.sqrt(var + EPS) * g_ref[...] + be_ref[...]
    hr = jnp.maximum(hn, 0.0)
    z = jnp.dot(hr.astype(jnp.bfloat16), w2_ref[...],
                preferred_element_type=jnp.float32) + b2_ref[...]
    zsq = _group_sum(z ** 2)
    dots = jnp.dot(z.astype(jnp.bfloat16), cbt_ref[...],
                   preferred_element_type=jnp.float32)
    lane = jax.lax.broadcasted_iota(jnp.int32, (BLK, 128), 1)
    minvec = None
    idxvec = None
    for c in range(K // 128):
        dch = zsq - 2.0 * dots[:, c * 128:(c + 1) * 128] \
            + csq_ref[:, c * 128:(c + 1) * 128]
        if c == 0:
            minvec = dch
            idxvec = lane
        else:
            better = dch < minvec
            idxvec = jnp.where(better, lane + c * 128, idxvec)
            minvec = jnp.where(better, dch, minvec)
    minv = jnp.min(minvec, axis=1)
    idx = jnp.min(jnp.where(minvec == minv[:, None], idxvec, K), axis=1)
    idx_ref[...] = idx[:, None]
    s = jnp.sum(minv).reshape(1, 1)

    @pl.when(i == 0)
    def _():
        loss_ref[...] = jnp.zeros_like(loss_ref)

    loss_ref[...] += s


_tc_call = pl.pallas_call(
    _tc_body,
    grid=(NUM_BLKS,),
    in_specs=[
        pl.BlockSpec((BLK, 2), lambda i: (i, 0)),
        pl.BlockSpec((2, H), lambda i: (0, 0)),
        pl.BlockSpec((1, H), lambda i: (0, 0)),
        pl.BlockSpec((1, H), lambda i: (0, 0)),
        pl.BlockSpec((1, H), lambda i: (0, 0)),
        pl.BlockSpec((H, D), lambda i: (0, 0)),
        pl.BlockSpec((1, D), lambda i: (0, 0)),
        pl.BlockSpec((D, K), lambda i: (0, 0)),
        pl.BlockSpec((K, D), lambda i: (0, 0)),
    ],
    out_specs=[
        pl.BlockSpec((BLK, 1), lambda i: (i, 0)),
        pl.BlockSpec((1, 1), lambda i: (0, 0)),
    ],
    out_shape=[
        jax.ShapeDtypeStruct((N, 1), jnp.int32),
        jax.ShapeDtypeStruct((1, 1), jnp.float32),
    ],
    scratch_shapes=[pltpu.VMEM((1, K), jnp.float32)],
)


SC_NCH = SC_BPW // SC_CH
SC_NBUF = 3


def _sc_gather_body(cb_hbm, idx_hbm, out_hbm, idx_v, b0, b1, b2,
                    g0, g1, g2, o0, o1, o2, cb_sh):
    sid = lax.axis_index("s")
    wid = sid * SC_NC + lax.axis_index("c")
    base = wid * SC_BPW
    bufs = (b0, b1, b2)
    gsem = (g0, g1, g2)
    osem = (o0, o1, o2)

    @pl.when(sid == 0)
    def _():
        pltpu.sync_copy(cb_hbm, cb_sh)

    pltpu.sync_copy(idx_hbm.at[pl.ds(base, SC_BPW)], idx_v)
    plsc.subcore_barrier()

    def gather(g):
        return pltpu.async_copy(
            cb_sh.at[idx_v.at[pl.ds(g * SC_CH, SC_CH)]],
            bufs[g % SC_NBUF], gsem[g % SC_NBUF])

    hg = {}
    ho = {}
    for g in range(SC_NBUF):
        hg[g] = gather(g)
    for g in range(SC_NCH):
        hg[g].wait()
        ho[g] = pltpu.async_copy(
            bufs[g % SC_NBUF],
            out_hbm.at[pl.ds(base + g * SC_CH, SC_CH)],
            osem[g % SC_NBUF])
        ng = g + SC_NBUF
        if ng < SC_NCH:
            ho[g].wait()
            hg[ng] = gather(ng)
    for g in range(SC_NCH - SC_NBUF, SC_NCH):
        if g >= 0 and g in ho and (g + SC_NBUF) >= SC_NCH:
            ho[g].wait()


@functools.lru_cache(maxsize=1)
def _make_sc_gather():
    return functools.partial(
        pl.kernel,
        mesh=plsc.VectorSubcoreMesh(core_axis_name="c", subcore_axis_name="s"),
        out_type=jax.ShapeDtypeStruct((N, 128), jnp.float32),
        scratch_types=[
            pltpu.VMEM((SC_BPW,), jnp.int32),
            pltpu.VMEM((SC_CH, 128), jnp.float32),
            pltpu.VMEM((SC_CH, 128), jnp.float32),
            pltpu.VMEM((SC_CH, 128), jnp.float32),
            pltpu.SemaphoreType.DMA,
            pltpu.SemaphoreType.DMA,
            pltpu.SemaphoreType.DMA,
            pltpu.SemaphoreType.DMA,
            pltpu.SemaphoreType.DMA,
            pltpu.SemaphoreType.DMA,
            pltpu.VMEM_SHARED((K, 128), jnp.float32),
        ],
    )(_sc_gather_body)


def kernel(xy, W1, b1, gamma, beta, W2, b2, codebook):
    f = xy.astype(jnp.float32)
    nx = f[:, 0] / 511.0 * 2.0 - 1.0
    ny = f[:, 1] / 511.0 * 2.0 - 1.0
    norm_xy = jnp.stack([nx, ny], axis=1)

    idx2d, loss_sum = _tc_call(
        norm_xy, W1, b1.reshape(1, H), gamma.reshape(1, H), beta.reshape(1, H),
        W2.astype(jnp.bfloat16), b2.reshape(1, D),
        codebook.T.astype(jnp.bfloat16), codebook)

    cb_pad = jnp.concatenate(
        [codebook, jnp.zeros((K, 128 - D), jnp.float32)], axis=1)
    q = _make_sc_gather()(cb_pad, idx2d.reshape(N))[:, :D]

    l = loss_sum[0, 0] / (N * D)
    loss = l + 0.25 * l
    return (q, idx2d, loss)


# speedup vs baseline: 2.2211x; 1.0856x over previous
import functools

import jax
import jax.numpy as jnp
from jax import lax
from jax.experimental import pallas as pl
from jax.experimental.pallas import tpu as pltpu
from jax.experimental.pallas import tpu_sc as plsc

N = 65536
H = 64
D = 64
K = 1024
EPS = 1e-5

BLK = 1024
NUM_BLKS = N // BLK

SC_NC = 2
SC_NS = 16
SC_NW = SC_NC * SC_NS
SC_BPW = N // SC_NW
SC_CH = 256


def _fold_sum(x):
    w = x.shape[-1]
    while w > 1:
        w //= 2
        x = x[:, :w] + x[:, w:2 * w]
    return x


def _group_sum(x):
    acc = x[:, 0:8]
    for v in range(1, 8):
        acc = acc + x[:, 8 * v:8 * v + 8]
    acc = acc[:, 0:4] + acc[:, 4:8]
    acc = acc[:, 0:2] + acc[:, 2:4]
    return acc[:, 0:1] + acc[:, 1:2]


def _tc_body(nxy_ref, w1_ref, b1_ref, g_ref, be_ref, w2_ref, b2_ref,
             cbt_ref, cb_ref, idx_ref, loss_ref, csq_ref):
    i = pl.program_id(0)

    @pl.when(i == 0)
    def _():
        csq_ref[...] = _group_sum(cb_ref[...] ** 2).reshape(1, K)

    nxy = nxy_ref[...]
    w1 = w1_ref[...]
    nb = nxy.astype(jnp.bfloat16).astype(jnp.float32)
    wb = w1.astype(jnp.bfloat16).astype(jnp.float32)
    h = (nb[:, 0:1] * wb[0:1, :] + nb[:, 1:2] * wb[1:2, :]) + b1_ref[...]
    mu = _fold_sum(h) / float(H)
    var = _group_sum((h - mu) ** 2) / float(H)
    hn = (h - mu) / jnp.sqrt(var + EPS) * g_ref[...] + be_ref[...]
    hr = jnp.maximum(hn, 0.0)
    z = jnp.dot(hr.astype(jnp.bfloat16), w2_ref[...],
                preferred_element_type=jnp.float32) + b2_ref[...]
    zsq = _group_sum(z ** 2)
    dots = jnp.dot(z.astype(jnp.bfloat16), cbt_ref[...],
                   preferred_element_type=jnp.float32)
    lane = jax.lax.broadcasted_iota(jnp.int32, (BLK, 128), 1)
    minvec = None
    idxvec = None
    for c in range(K // 128):
        dch = zsq - 2.0 * dots[:, c * 128:(c + 1) * 128] \
            + csq_ref[:, c * 128:(c + 1) * 128]
        if c == 0:
            minvec = dch
            idxvec = lane
        else:
            better = dch < minvec
            idxvec = jnp.where(better, lane + c * 128, idxvec)
            minvec = jnp.where(better, dch, minvec)
    minv = jnp.min(minvec, axis=1)
    idx = jnp.min(jnp.where(minvec == minv[:, None], idxvec, K), axis=1)
    idx_ref[...] = idx[:, None]
    s = jnp.sum(minv).reshape(1, 1)

    @pl.when(i == 0)
    def _():
        loss_ref[...] = jnp.zeros_like(loss_ref)

    loss_ref[...] += s


_tc_call = pl.pallas_call(
    _tc_body,
    grid=(NUM_BLKS,),
    in_specs=[
        pl.BlockSpec((BLK, 2), lambda i: (i, 0)),
        pl.BlockSpec((2, H), lambda i: (0, 0)),
        pl.BlockSpec((1, H), lambda i: (0, 0)),
        pl.BlockSpec((1, H), lambda i: (0, 0)),
        pl.BlockSpec((1, H), lambda i: (0, 0)),
        pl.BlockSpec((H, D), lambda i: (0, 0)),
        pl.BlockSpec((1, D), lambda i: (0, 0)),
        pl.BlockSpec((D, K), lambda i: (0, 0)),
        pl.BlockSpec((K, D), lambda i: (0, 0)),
    ],
    out_specs=[
        pl.BlockSpec((BLK, 1), lambda i: (i, 0)),
        pl.BlockSpec((1, 1), lambda i: (0, 0)),
    ],
    out_shape=[
        jax.ShapeDtypeStruct((N, 1), jnp.int32),
        jax.ShapeDtypeStruct((1, 1), jnp.float32),
    ],
    scratch_shapes=[pltpu.VMEM((1, K), jnp.float32)],
)


SC_NCH = SC_BPW // SC_CH
SC_NBUF = 3


def _sc_gather_body(cb_hbm, idx_hbm, out_hbm, idx_v, b0, b1, b2,
                    g0, g1, g2, o0, o1, o2, cb_sh):
    sid = lax.axis_index("s")
    wid = sid * SC_NC + lax.axis_index("c")
    base = wid * SC_BPW
    bufs = (b0, b1, b2)
    gsem = (g0, g1, g2)
    osem = (o0, o1, o2)

    @pl.when(sid == 0)
    def _():
        pltpu.sync_copy(cb_hbm, cb_sh)

    pltpu.sync_copy(idx_hbm.at[pl.ds(base, SC_BPW)], idx_v)
    plsc.subcore_barrier()

    def gather(g):
        return pltpu.async_copy(
            cb_sh.at[idx_v.at[pl.ds(g * SC_CH, SC_CH)]],
            bufs[g % SC_NBUF], gsem[g % SC_NBUF])

    hg = {}
    ho = {}
    for g in range(SC_NBUF):
        hg[g] = gather(g)
    for g in range(SC_NCH):
        hg[g].wait()
        ho[g] = pltpu.async_copy(
            bufs[g % SC_NBUF],
            out_hbm.at[pl.ds(base + g * SC_CH, SC_CH)],
            osem[g % SC_NBUF])
        ng = g + SC_NBUF
        if ng < SC_NCH:
            ho[g].wait()
            hg[ng] = gather(ng)
    for g in range(SC_NCH - SC_NBUF, SC_NCH):
        if g >= 0 and g in ho and (g + SC_NBUF) >= SC_NCH:
            ho[g].wait()


@functools.lru_cache(maxsize=1)
def _make_sc_gather():
    return functools.partial(
        pl.kernel,
        mesh=plsc.VectorSubcoreMesh(core_axis_name="c", subcore_axis_name="s"),
        out_type=jax.ShapeDtypeStruct((N, 128), jnp.float32),
        scratch_types=[
            pltpu.VMEM((SC_BPW,), jnp.int32),
            pltpu.VMEM((SC_CH, 128), jnp.float32),
            pltpu.VMEM((SC_CH, 128), jnp.float32),
            pltpu.VMEM((SC_CH, 128), jnp.float32),
            pltpu.SemaphoreType.DMA,
            pltpu.SemaphoreType.DMA,
            pltpu.SemaphoreType.DMA,
            pltpu.SemaphoreType.DMA,
            pltpu.SemaphoreType.DMA,
            pltpu.SemaphoreType.DMA,
            pltpu.VMEM_SHARED((K, 128), jnp.float32),
        ],
    )(_sc_gather_body)


def kernel(xy, W1, b1, gamma, beta, W2, b2, codebook):
    f = xy.astype(jnp.float32)
    nx = f[:, 0] / 511.0 * 2.0 - 1.0
    ny = f[:, 1] / 511.0 * 2.0 - 1.0
    norm_xy = jnp.stack([nx, ny], axis=1)

    idx2d, loss_sum = _tc_call(
        norm_xy, W1, b1.reshape(1, H), gamma.reshape(1, H), beta.reshape(1, H),
        W2.astype(jnp.bfloat16), b2.reshape(1, D),
        codebook.T.astype(jnp.bfloat16), codebook)

    cb_pad = jnp.concatenate(
        [codebook, jnp.zeros((K, 128 - D), jnp.float32)], axis=1)
    q = _make_sc_gather()(cb_pad, idx2d.reshape(N))[:, :D]

    l = loss_sum[0, 0] / (N * D)
    loss = l + 0.25 * l
    return (q, idx2d, loss)


# speedup vs baseline: 2.2478x; 1.0120x over previous
import functools

import jax
import jax.numpy as jnp
from jax import lax
from jax.experimental import pallas as pl
from jax.experimental.pallas import tpu as pltpu
from jax.experimental.pallas import tpu_sc as plsc

N = 65536
H = 64
D = 64
K = 1024
EPS = 1e-5

BLK = 2048
NUM_BLKS = N // BLK

SC_NC = 2
SC_NS = 16
SC_NW = SC_NC * SC_NS
SC_BPW = N // SC_NW
SC_CH = 256


def _fold_sum(x):
    w = x.shape[-1]
    while w > 1:
        w //= 2
        x = x[:, :w] + x[:, w:2 * w]
    return x


def _group_sum(x):
    acc = x[:, 0:8]
    for v in range(1, 8):
        acc = acc + x[:, 8 * v:8 * v + 8]
    acc = acc[:, 0:4] + acc[:, 4:8]
    acc = acc[:, 0:2] + acc[:, 2:4]
    return acc[:, 0:1] + acc[:, 1:2]


def _tc_body(nxy_ref, w1_ref, b1_ref, g_ref, be_ref, w2_ref, b2_ref,
             cbt_ref, cb_ref, idx_ref, loss_ref, csq_ref):
    i = pl.program_id(0)

    @pl.when(i == 0)
    def _():
        csq_ref[...] = _group_sum(cb_ref[...] ** 2).reshape(1, K)

    nxy = nxy_ref[...]
    w1 = w1_ref[...]
    nb = nxy.astype(jnp.bfloat16).astype(jnp.float32)
    wb = w1.astype(jnp.bfloat16).astype(jnp.float32)
    h = (nb[:, 0:1] * wb[0:1, :] + nb[:, 1:2] * wb[1:2, :]) + b1_ref[...]
    mu = _fold_sum(h) / float(H)
    var = _group_sum((h - mu) ** 2) / float(H)
    hn = (h - mu) / jnp.sqrt(var + EPS) * g_ref[...] + be_ref[...]
    hr = jnp.maximum(hn, 0.0)
    z = jnp.dot(hr.astype(jnp.bfloat16), w2_ref[...],
                preferred_element_type=jnp.float32) + b2_ref[...]
    zsq = _group_sum(z ** 2)
    dots = jnp.dot(z.astype(jnp.bfloat16), cbt_ref[...],
                   preferred_element_type=jnp.float32)
    lane = jax.lax.broadcasted_iota(jnp.int32, (BLK, 128), 1)
    minvec = None
    idxvec = None
    for c in range(K // 128):
        dch = zsq - 2.0 * dots[:, c * 128:(c + 1) * 128] \
            + csq_ref[:, c * 128:(c + 1) * 128]
        if c == 0:
            minvec = dch
            idxvec = lane
        else:
            better = dch < minvec
            idxvec = jnp.where(better, lane + c * 128, idxvec)
            minvec = jnp.where(better, dch, minvec)
    minv = jnp.min(minvec, axis=1)
    idx = jnp.min(jnp.where(minvec == minv[:, None], idxvec, K), axis=1)
    idx_ref[...] = idx[:, None]
    s = jnp.sum(minv).reshape(1, 1)

    @pl.when(i == 0)
    def _():
        loss_ref[...] = jnp.zeros_like(loss_ref)

    loss_ref[...] += s


_tc_call = pl.pallas_call(
    _tc_body,
    grid=(NUM_BLKS,),
    in_specs=[
        pl.BlockSpec((BLK, 2), lambda i: (i, 0)),
        pl.BlockSpec((2, H), lambda i: (0, 0)),
        pl.BlockSpec((1, H), lambda i: (0, 0)),
        pl.BlockSpec((1, H), lambda i: (0, 0)),
        pl.BlockSpec((1, H), lambda i: (0, 0)),
        pl.BlockSpec((H, D), lambda i: (0, 0)),
        pl.BlockSpec((1, D), lambda i: (0, 0)),
        pl.BlockSpec((D, K), lambda i: (0, 0)),
        pl.BlockSpec((K, D), lambda i: (0, 0)),
    ],
    out_specs=[
        pl.BlockSpec((BLK, 1), lambda i: (i, 0)),
        pl.BlockSpec((1, 1), lambda i: (0, 0)),
    ],
    out_shape=[
        jax.ShapeDtypeStruct((N, 1), jnp.int32),
        jax.ShapeDtypeStruct((1, 1), jnp.float32),
    ],
    scratch_shapes=[pltpu.VMEM((1, K), jnp.float32)],
)


SC_NCH = SC_BPW // SC_CH
SC_NBUF = 3


def _sc_gather_body(cb_hbm, idx_hbm, out_hbm, idx_v, b0, b1, b2,
                    g0, g1, g2, o0, o1, o2, cb_sh):
    sid = lax.axis_index("s")
    wid = sid * SC_NC + lax.axis_index("c")
    base = wid * SC_BPW
    bufs = (b0, b1, b2)
    gsem = (g0, g1, g2)
    osem = (o0, o1, o2)

    @pl.when(sid == 0)
    def _():
        pltpu.sync_copy(cb_hbm, cb_sh)

    pltpu.sync_copy(idx_hbm.at[pl.ds(base, SC_BPW)], idx_v)
    plsc.subcore_barrier()

    def gather(g):
        return pltpu.async_copy(
            cb_sh.at[idx_v.at[pl.ds(g * SC_CH, SC_CH)]],
            bufs[g % SC_NBUF], gsem[g % SC_NBUF])

    hg = {}
    ho = {}
    for g in range(SC_NBUF):
        hg[g] = gather(g)
    for g in range(SC_NCH):
        hg[g].wait()
        ho[g] = pltpu.async_copy(
            bufs[g % SC_NBUF],
            out_hbm.at[pl.ds(base + g * SC_CH, SC_CH)],
            osem[g % SC_NBUF])
        ng = g + SC_NBUF
        if ng < SC_NCH:
            ho[g].wait()
            hg[ng] = gather(ng)
    for g in range(SC_NCH - SC_NBUF, SC_NCH):
        if g >= 0 and g in ho and (g + SC_NBUF) >= SC_NCH:
            ho[g].wait()


@functools.lru_cache(maxsize=1)
def _make_sc_gather():
    return functools.partial(
        pl.kernel,
        mesh=plsc.VectorSubcoreMesh(core_axis_name="c", subcore_axis_name="s"),
        out_type=jax.ShapeDtypeStruct((N, 128), jnp.float32),
        scratch_types=[
            pltpu.VMEM((SC_BPW,), jnp.int32),
            pltpu.VMEM((SC_CH, 128), jnp.float32),
            pltpu.VMEM((SC_CH, 128), jnp.float32),
            pltpu.VMEM((SC_CH, 128), jnp.float32),
            pltpu.SemaphoreType.DMA,
            pltpu.SemaphoreType.DMA,
            pltpu.SemaphoreType.DMA,
            pltpu.SemaphoreType.DMA,
            pltpu.SemaphoreType.DMA,
            pltpu.SemaphoreType.DMA,
            pltpu.VMEM_SHARED((K, 128), jnp.float32),
        ],
    )(_sc_gather_body)


def kernel(xy, W1, b1, gamma, beta, W2, b2, codebook):
    f = xy.astype(jnp.float32)
    nx = f[:, 0] / 511.0 * 2.0 - 1.0
    ny = f[:, 1] / 511.0 * 2.0 - 1.0
    norm_xy = jnp.stack([nx, ny], axis=1)

    idx2d, loss_sum = _tc_call(
        norm_xy, W1, b1.reshape(1, H), gamma.reshape(1, H), beta.reshape(1, H),
        W2.astype(jnp.bfloat16), b2.reshape(1, D),
        codebook.T.astype(jnp.bfloat16), codebook)

    cb_pad = jnp.concatenate(
        [codebook, jnp.zeros((K, 128 - D), jnp.float32)], axis=1)
    q = _make_sc_gather()(cb_pad, idx2d.reshape(N))[:, :D]

    l = loss_sum[0, 0] / (N * D)
    loss = l + 0.25 * l
    return (q, idx2d, loss)
